# baseline (speedup 1.0000x reference)
import functools

import jax
import jax.numpy as jnp
from jax.experimental import pallas as pl
from jax.experimental.pallas import tpu as pltpu

_LANES = 128
_TILE = 20000


def _idloss_body(pred_ref, tid_ref, out_ref, acc, acc_g, acc_g2, *, nsteps):
    step = pl.program_id(0)

    @pl.when(step == 0)
    def _init():
        acc[...] = jnp.zeros_like(acc)
        acc_g[...] = jnp.zeros_like(acc_g)
        acc_g2[...] = jnp.zeros_like(acc_g2)

    x = pred_ref[...]
    tid = tid_ref[0]
    t = x.shape[0]
    dn = (((1,), (0,)), ((), ()))
    xb = x.astype(jnp.bfloat16)
    sub_t = jax.lax.broadcasted_iota(jnp.int16, (_LANES, t), 0)
    tid_b = tid.astype(jnp.int16)
    onehot_t = jnp.where(tid_b == sub_t, jnp.bfloat16(1), jnp.bfloat16(0))
    acc_g[...] += jax.lax.dot_general(
        onehot_t, xb, dn, preferred_element_type=jnp.float32)
    acc_g2[...] += jax.lax.dot_general(
        onehot_t, xb * xb, dn, preferred_element_type=jnp.float32)
    acc[...] += jax.lax.dot_general(
        onehot_t, jnp.ones((t, 1), jnp.bfloat16), dn,
        preferred_element_type=jnp.float32)

    @pl.when(step == nsteps - 1)
    def _tail():
        c = jnp.float32(_LANES)
        counts_c = acc[...]
        sums_c = jnp.sum(acc_g[...], axis=1, keepdims=True)
        sumsq_c = jnp.sum(acc_g2[...], axis=1, keepdims=True)
        nelem_c = counts_c * c
        mean_c = sums_c / jnp.maximum(nelem_c, 1.0)
        var_c = (sumsq_c - nelem_c * mean_c * mean_c) / jnp.maximum(
            nelem_c - 1.0, 1.0)
        std_c = jnp.where(
            counts_c > 1.0, jnp.sqrt(jnp.maximum(var_c, 0.0)), 0.0)
        mean = jnp.transpose(mean_c)
        std = jnp.transpose(std_c)

        d = mean - mean_c
        nrm = jnp.sqrt(c * d * d)
        a = jnp.abs(d) / (nrm + 1e-5)
        p = a * std
        lane = jax.lax.broadcasted_iota(jnp.int32, (1, _LANES), 1)

        def k_step(k, acc):
            sel = (lane == k).astype(jnp.float32)
            m_k = jnp.sum(sel * mean)
            s_k = jnp.sum(sel * std)
            dk = mean - m_k
            nk = jnp.sqrt(c * dk * dk) + 1e-5
            num = p + a * s_k
            f = num / (nk + num)
            keep = (lane < k).astype(jnp.float32)
            return acc + f * keep

        acc2d = jax.lax.fori_loop(
            0, _LANES, k_step, jnp.zeros((_LANES, _LANES), jnp.float32))
        out_ref[...] = jnp.reshape(
            jnp.sum(acc2d) / jnp.float32(_LANES ** 3), (1, 1))


@jax.jit
def kernel(pred_id, target_id):
    n, c = pred_id.shape
    tile = min(_TILE, n)
    nsteps = -(-n // tile)
    npad = nsteps * tile - n
    if npad:
        pred_id = jnp.pad(pred_id, ((0, npad), (0, 0)))
        target_id = jnp.pad(target_id, (0, npad), constant_values=_LANES - 1)
    tid3 = target_id.reshape(nsteps, 1, tile)

    out = pl.pallas_call(
        functools.partial(_idloss_body, nsteps=nsteps),
        grid=(nsteps,),
        in_specs=[
            pl.BlockSpec((tile, c), lambda i: (i, 0)),
            pl.BlockSpec((1, 1, tile), lambda i: (i, 0, 0)),
        ],
        out_specs=pl.BlockSpec((1, 1), lambda i: (0, 0)),
        out_shape=jax.ShapeDtypeStruct((1, 1), jnp.float32),
        scratch_shapes=[
            pltpu.VMEM((_LANES, 1), jnp.float32),
            pltpu.VMEM((_LANES, _LANES), jnp.float32),
            pltpu.VMEM((_LANES, _LANES), jnp.float32),
        ],
        compiler_params=pltpu.CompilerParams(
            dimension_semantics=("arbitrary",)),
    )(pred_id, tid3)
    return out[0, 0]


# speedup vs baseline: 1.0019x; 1.0019x over previous
import functools

import jax
import jax.numpy as jnp
from jax.experimental import pallas as pl
from jax.experimental.pallas import tpu as pltpu

_LANES = 128
_TILE = 16000


def _idloss_body(pred_ref, tid_ref, out_ref, acc, acc_g, acc_g2, *, nsteps):
    step = pl.program_id(0)

    @pl.when(step == 0)
    def _init():
        acc[...] = jnp.zeros_like(acc)
        acc_g[...] = jnp.zeros_like(acc_g)
        acc_g2[...] = jnp.zeros_like(acc_g2)

    x = pred_ref[...]
    tid = tid_ref[0]
    t = x.shape[0]
    dn = (((1,), (0,)), ((), ()))
    xb = x.astype(jnp.bfloat16)
    sub_t = jax.lax.broadcasted_iota(jnp.int16, (_LANES, t), 0)
    tid_b = tid.astype(jnp.int16)
    onehot_t = jnp.where(tid_b == sub_t, jnp.bfloat16(1), jnp.bfloat16(0))
    acc_g[...] += jax.lax.dot_general(
        onehot_t, xb, dn, preferred_element_type=jnp.float32)
    acc_g2[...] += jax.lax.dot_general(
        onehot_t, xb * xb, dn, preferred_element_type=jnp.float32)
    acc[...] += jax.lax.dot_general(
        onehot_t, jnp.ones((t, 1), jnp.bfloat16), dn,
        preferred_element_type=jnp.float32)

    @pl.when(step == nsteps - 1)
    def _tail():
        c = jnp.float32(_LANES)
        counts_c = acc[...]
        sums_c = jnp.sum(acc_g[...], axis=1, keepdims=True)
        sumsq_c = jnp.sum(acc_g2[...], axis=1, keepdims=True)
        nelem_c = counts_c * c
        mean_c = sums_c / jnp.maximum(nelem_c, 1.0)
        var_c = (sumsq_c - nelem_c * mean_c * mean_c) / jnp.maximum(
            nelem_c - 1.0, 1.0)
        std_c = jnp.where(
            counts_c > 1.0, jnp.sqrt(jnp.maximum(var_c, 0.0)), 0.0)
        mean = jnp.transpose(mean_c)
        std = jnp.transpose(std_c)

        d = mean - mean_c
        nrm = jnp.sqrt(c * d * d)
        a = jnp.abs(d) / (nrm + 1e-5)
        p = a * std
        lane = jax.lax.broadcasted_iota(jnp.int32, (1, _LANES), 1)

        def k_step(k, acc):
            sel = (lane == k).astype(jnp.float32)
            m_k = jnp.sum(sel * mean)
            s_k = jnp.sum(sel * std)
            dk = mean - m_k
            nk = jnp.sqrt(c * dk * dk) + 1e-5
            num = p + a * s_k
            f = num / (nk + num)
            keep = (lane < k).astype(jnp.float32)
            return acc + f * keep

        acc2d = jax.lax.fori_loop(
            0, _LANES, k_step, jnp.zeros((_LANES, _LANES), jnp.float32))
        out_ref[...] = jnp.reshape(
            jnp.sum(acc2d) / jnp.float32(_LANES ** 3), (1, 1))


@jax.jit
def kernel(pred_id, target_id):
    n, c = pred_id.shape
    tile = min(_TILE, n)
    nsteps = -(-n // tile)
    npad = nsteps * tile - n
    if npad:
        pred_id = jnp.pad(pred_id, ((0, npad), (0, 0)))
        target_id = jnp.pad(target_id, (0, npad), constant_values=_LANES - 1)
    tid3 = target_id.reshape(nsteps, 1, tile)

    out = pl.pallas_call(
        functools.partial(_idloss_body, nsteps=nsteps),
        grid=(nsteps,),
        in_specs=[
            pl.BlockSpec((tile, c), lambda i: (i, 0)),
            pl.BlockSpec((1, 1, tile), lambda i: (i, 0, 0)),
        ],
        out_specs=pl.BlockSpec((1, 1), lambda i: (0, 0)),
        out_shape=jax.ShapeDtypeStruct((1, 1), jnp.float32),
        scratch_shapes=[
            pltpu.VMEM((_LANES, 1), jnp.float32),
            pltpu.VMEM((_LANES, _LANES), jnp.float32),
            pltpu.VMEM((_LANES, _LANES), jnp.float32),
        ],
        compiler_params=pltpu.CompilerParams(
            dimension_semantics=("arbitrary",)),
    )(pred_id, tid3)
    return out[0, 0]
